# baseline (speedup 1.0000x reference)
import functools

import jax
import jax.numpy as jnp
from jax import lax
from jax.experimental import pallas as pl
from jax.experimental.pallas import tpu as pltpu
from jax.experimental.pallas import tpu_sc as plsc

H = 16
NB = 32
QL = 2048
KL = 2048
M = 4096
PADW = 4352
BQ = 256
NQ = 8

_THR = (12, 16, 23, 32, 46, 64, 91)


def _sc_lookup(Wc0):
    info = plsc.get_sparse_core_info()
    ns, L = info.num_subcores, info.num_lanes
    bpw = M // ns

    mesh = plsc.VectorSubcoreMesh(
        core_axis_name="c", subcore_axis_name="s", num_cores=1
    )

    @functools.partial(
        pl.kernel,
        mesh=mesh,
        out_type=jax.ShapeDtypeStruct((M,), jnp.float32),
        scratch_types=[
            pltpu.VMEM(Wc0.shape, jnp.float32),
            pltpu.VMEM((bpw,), jnp.float32),
        ],
    )
    def k(w_hbm, t_hbm, w_v, td_v):
        base = lax.axis_index("s") * bpw
        pltpu.sync_copy(w_hbm, w_v)
        w_lo = w_v[pl.ds(0, L)]
        w_hi = w_v[pl.ds(L, L)]
        for j in range(bpw // L):
            m = lax.iota(jnp.int32, L) + (base + j * L)
            d = m - 2047
            a = jnp.abs(d)
            large = jnp.full((L,), 8, jnp.int32)
            for t in _THR:
                large = large + jnp.where(a >= t, 1, 0).astype(jnp.int32)
            b = jnp.where(a < 8, a, large).astype(jnp.int32)
            lo = w_lo.at[b].get(mode="promise_in_bounds")
            hi = w_hi.at[b].get(mode="promise_in_bounds")
            td_v[pl.ds(j * L, L)] = jnp.where(d > 0, hi, lo)
        pltpu.sync_copy(td_v, t_hbm.at[pl.ds(base, bpw)])

    return k(Wc0)


def _bucket_map():
    j = lax.broadcasted_iota(jnp.int32, (1, M), 1)
    d = j - 2047
    a = jnp.abs(d)
    rb = jnp.where(d > 0, 16, 0).astype(jnp.int32)
    large = jnp.full((1, M), 8, jnp.int32)
    for t in _THR:
        large = large + jnp.where(a >= t, 1, 0).astype(jnp.int32)
    return rb + jnp.where(a < 8, a, large).astype(jnp.int32)


def _tc_big_body(wt_ref, o_ref, tt16_ref, tp8_ref, t128_ref, sems):
    hp = pl.program_id(0)
    h = hp + 1
    t = pl.program_id(1)
    step = hp * (QL // BQ) + t
    buf = lax.rem(hp, 2)

    @pl.when((hp == 0) & (t == 0))
    def _table():
        bk = _bucket_map()
        acc = jnp.zeros((H, M), jnp.float32)
        for b in range(NB):
            acc = jnp.where(bk == b, wt_ref[:, b : b + 1], acc)
        tt16_ref[...] = acc

    @pl.when(t == 0)
    def _build():
        for b in range(8):
            tp8_ref[b, pl.ds(b, M)] = tt16_ref[h, :]
        for a in range(BQ // 8):
            t128_ref[buf, 8 * a : 8 * a + 8, :] = tp8_ref[
                :, (BQ - 1) - 8 * a : (BQ - 1) - 8 * a + M
            ]

    off = pl.multiple_of((KL - BQ) - BQ * t, 128)
    src = t128_ref.at[buf, :, pl.ds(off, KL)]
    dst = o_ref.at[h, pl.ds(t * BQ, BQ), :]
    slot = lax.rem(step, NQ)

    @pl.when(step >= NQ)
    def _drain_slot():
        pltpu.make_async_copy(src, dst, sems.at[slot]).wait()

    pltpu.make_async_copy(src, dst, sems.at[slot]).start()

    @pl.when(step == (H - 1) * (QL // BQ) - 1)
    def _drain_all():
        for q in range(NQ):
            pltpu.make_async_copy(src, dst, sems.at[q]).wait()


def _tc_big(Wt):
    return pl.pallas_call(
        _tc_big_body,
        grid=(H - 1, QL // BQ),
        in_specs=[pl.BlockSpec((H, NB), lambda hp, t: (0, 0))],
        out_specs=pl.BlockSpec(memory_space=pl.ANY),
        out_shape=jax.ShapeDtypeStruct((H, QL, KL), jnp.float32),
        scratch_shapes=[
            pltpu.VMEM((H, M), jnp.float32),
            pltpu.VMEM((8, PADW), jnp.float32),
            pltpu.VMEM((2, BQ, M), jnp.float32),
            pltpu.SemaphoreType.DMA((NQ,)),
        ],
        compiler_params=pltpu.CompilerParams(
            dimension_semantics=("arbitrary", "arbitrary"),
        ),
    )(Wt)


def _tc_head0_body(o_in_ref, td0_ref, o_ref, t128_ref, tp8_ref, sems):
    del o_in_ref
    t = pl.program_id(0)

    @pl.when(t == 0)
    def _build():
        for b in range(8):
            tp8_ref[b, pl.ds(b, M)] = td0_ref[0, :]
        for a in range(16):
            t128_ref[8 * a : 8 * a + 8, :] = tp8_ref[
                :, 127 - 8 * a : 127 - 8 * a + M
            ]

    off = pl.multiple_of(1920 - 128 * t, 128)
    src = t128_ref.at[:, pl.ds(off, KL)]
    dst = o_ref.at[0, pl.ds(t * 128, 128), :]
    slot = lax.rem(t, NQ)

    @pl.when(t >= NQ)
    def _drain_slot():
        pltpu.make_async_copy(src, dst, sems.at[slot]).wait()

    pltpu.make_async_copy(src, dst, sems.at[slot]).start()

    @pl.when(t == QL // 128 - 1)
    def _drain_all():
        for q in range(NQ):
            pltpu.make_async_copy(src, dst, sems.at[q]).wait()


def _tc_head0(out0, td0):
    return pl.pallas_call(
        _tc_head0_body,
        grid=(QL // 128,),
        in_specs=[
            pl.BlockSpec(memory_space=pl.ANY),
            pl.BlockSpec((1, M), lambda t: (0, 0)),
        ],
        out_specs=pl.BlockSpec(memory_space=pl.ANY),
        out_shape=jax.ShapeDtypeStruct((H, QL, KL), jnp.float32),
        input_output_aliases={0: 0},
        scratch_shapes=[
            pltpu.VMEM((128, M), jnp.float32),
            pltpu.VMEM((8, PADW), jnp.float32),
            pltpu.SemaphoreType.DMA((NQ,)),
        ],
        compiler_params=pltpu.CompilerParams(
            dimension_semantics=("arbitrary",),
        ),
    )(out0, td0)


def kernel(query_length, key_length, W):
    del query_length, key_length
    W = W.astype(jnp.float32)

    td0 = _sc_lookup(W[:, 0])

    out0 = _tc_big(W.T)

    out = _tc_head0(out0, td0.reshape(1, M))
    return out[None]


# speedup vs baseline: 1.0144x; 1.0144x over previous
import functools

import jax
import jax.numpy as jnp
from jax import lax
from jax.experimental import pallas as pl
from jax.experimental.pallas import tpu as pltpu
from jax.experimental.pallas import tpu_sc as plsc

H = 16
NB = 32
QL = 2048
KL = 2048
M = 4096
PADW = 4352
BQ = 128
NQ = 8

_THR = (12, 16, 23, 32, 46, 64, 91)


def _sc_lookup(Wc0):
    info = plsc.get_sparse_core_info()
    ns, L = info.num_subcores, info.num_lanes
    bpw = M // ns

    mesh = plsc.VectorSubcoreMesh(
        core_axis_name="c", subcore_axis_name="s", num_cores=1
    )

    @functools.partial(
        pl.kernel,
        mesh=mesh,
        out_type=jax.ShapeDtypeStruct((M,), jnp.float32),
        scratch_types=[
            pltpu.VMEM(Wc0.shape, jnp.float32),
            pltpu.VMEM((bpw,), jnp.float32),
        ],
    )
    def k(w_hbm, t_hbm, w_v, td_v):
        base = lax.axis_index("s") * bpw
        pltpu.sync_copy(w_hbm, w_v)
        w_lo = w_v[pl.ds(0, L)]
        w_hi = w_v[pl.ds(L, L)]
        for j in range(bpw // L):
            m = lax.iota(jnp.int32, L) + (base + j * L)
            d = m - 2047
            a = jnp.abs(d)
            large = jnp.full((L,), 8, jnp.int32)
            for t in _THR:
                large = large + jnp.where(a >= t, 1, 0).astype(jnp.int32)
            b = jnp.where(a < 8, a, large).astype(jnp.int32)
            lo = w_lo.at[b].get(mode="promise_in_bounds")
            hi = w_hi.at[b].get(mode="promise_in_bounds")
            td_v[pl.ds(j * L, L)] = jnp.where(d > 0, hi, lo)
        pltpu.sync_copy(td_v, t_hbm.at[pl.ds(base, bpw)])

    return k(Wc0)


def _bucket_map():
    j = lax.broadcasted_iota(jnp.int32, (1, M), 1)
    d = j - 2047
    a = jnp.abs(d)
    rb = jnp.where(d > 0, 16, 0).astype(jnp.int32)
    large = jnp.full((1, M), 8, jnp.int32)
    for t in _THR:
        large = large + jnp.where(a >= t, 1, 0).astype(jnp.int32)
    return rb + jnp.where(a < 8, a, large).astype(jnp.int32)


def _tc_big_body(wt_ref, o_ref, tt16_ref, tp8_ref, t128_ref, sems):
    hp = pl.program_id(0)
    h = hp + 1
    t = pl.program_id(1)
    step = hp * (QL // BQ) + t
    buf = lax.rem(hp, 2)

    @pl.when((hp == 0) & (t == 0))
    def _table():
        bk = _bucket_map()
        acc = jnp.zeros((H, M), jnp.float32)
        for b in range(NB):
            acc = jnp.where(bk == b, wt_ref[:, b : b + 1], acc)
        tt16_ref[...] = acc

    @pl.when(t == 0)
    def _build():
        for b in range(8):
            tp8_ref[b, pl.ds(b, M)] = tt16_ref[h, :]
        for a in range(BQ // 8):
            t128_ref[buf, 8 * a : 8 * a + 8, :] = tp8_ref[
                :, (BQ - 1) - 8 * a : (BQ - 1) - 8 * a + M
            ]

    off = pl.multiple_of((KL - BQ) - BQ * t, 128)
    src = t128_ref.at[buf, :, pl.ds(off, KL)]
    dst = o_ref.at[h, pl.ds(t * BQ, BQ), :]
    slot = lax.rem(step, NQ)

    @pl.when(step >= NQ)
    def _drain_slot():
        pltpu.make_async_copy(src, dst, sems.at[slot]).wait()

    pltpu.make_async_copy(src, dst, sems.at[slot]).start()

    @pl.when(step == (H - 1) * (QL // BQ) - 1)
    def _drain_all():
        for q in range(NQ):
            pltpu.make_async_copy(src, dst, sems.at[q]).wait()


def _tc_big(Wt):
    return pl.pallas_call(
        _tc_big_body,
        grid=(H - 1, QL // BQ),
        in_specs=[pl.BlockSpec((H, NB), lambda hp, t: (0, 0))],
        out_specs=pl.BlockSpec(memory_space=pl.ANY),
        out_shape=jax.ShapeDtypeStruct((H, QL, KL), jnp.float32),
        scratch_shapes=[
            pltpu.VMEM((H, M), jnp.float32),
            pltpu.VMEM((8, PADW), jnp.float32),
            pltpu.VMEM((2, BQ, M), jnp.float32),
            pltpu.SemaphoreType.DMA((NQ,)),
        ],
        compiler_params=pltpu.CompilerParams(
            dimension_semantics=("arbitrary", "arbitrary"),
        ),
    )(Wt)


def _tc_head0_body(o_in_ref, td0_ref, o_ref, t128_ref, tp8_ref, sems):
    del o_in_ref
    t = pl.program_id(0)

    @pl.when(t == 0)
    def _build():
        for b in range(8):
            tp8_ref[b, pl.ds(b, M)] = td0_ref[0, :]
        for a in range(16):
            t128_ref[8 * a : 8 * a + 8, :] = tp8_ref[
                :, 127 - 8 * a : 127 - 8 * a + M
            ]

    off = pl.multiple_of(1920 - 128 * t, 128)
    src = t128_ref.at[:, pl.ds(off, KL)]
    dst = o_ref.at[0, pl.ds(t * 128, 128), :]
    slot = lax.rem(t, NQ)

    @pl.when(t >= NQ)
    def _drain_slot():
        pltpu.make_async_copy(src, dst, sems.at[slot]).wait()

    pltpu.make_async_copy(src, dst, sems.at[slot]).start()

    @pl.when(t == QL // 128 - 1)
    def _drain_all():
        for q in range(NQ):
            pltpu.make_async_copy(src, dst, sems.at[q]).wait()


def _tc_head0(out0, td0):
    return pl.pallas_call(
        _tc_head0_body,
        grid=(QL // 128,),
        in_specs=[
            pl.BlockSpec(memory_space=pl.ANY),
            pl.BlockSpec((1, M), lambda t: (0, 0)),
        ],
        out_specs=pl.BlockSpec(memory_space=pl.ANY),
        out_shape=jax.ShapeDtypeStruct((H, QL, KL), jnp.float32),
        input_output_aliases={0: 0},
        scratch_shapes=[
            pltpu.VMEM((128, M), jnp.float32),
            pltpu.VMEM((8, PADW), jnp.float32),
            pltpu.SemaphoreType.DMA((NQ,)),
        ],
        compiler_params=pltpu.CompilerParams(
            dimension_semantics=("arbitrary",),
        ),
    )(out0, td0)


def kernel(query_length, key_length, W):
    del query_length, key_length
    W = W.astype(jnp.float32)

    td0 = _sc_lookup(W[:, 0])

    out0 = _tc_big(W.T)

    out = _tc_head0(out0, td0.reshape(1, M))
    return out[None]
